# baseline (speedup 1.0000x reference)
import functools

import jax
import jax.numpy as jnp
from jax import lax
from jax.experimental import pallas as pl
from jax.experimental.pallas import tpu as pltpu
from jax.experimental.pallas import tpu_sc as plsc

NODES = 10000
JUNK_ROWS = 64
ACC_ROWS = NODES + JUNK_ROWS
FEAT = 128
EDGES = 320000
GRAPHS = 16

CORES = 2
SUBCORES = 16
TILES = CORES * SUBCORES

CHUNK = 96
NCH = 106
EPT = NCH * CHUNK
EDGES_PAD = EPT * TILES
NBUF = 2
NGRP = NCH // NBUF

RBLK = 2000
GRID = NODES // RBLK


def _sc_mesh():
    return plsc.VectorSubcoreMesh(core_axis_name="c", subcore_axis_name="s")


def _sc_degree(dst_r, zeros_acc):
    @functools.partial(
        pl.kernel,
        mesh=_sc_mesh(),
        out_type=(
            jax.ShapeDtypeStruct((ACC_ROWS, FEAT), jnp.float32),
            jax.ShapeDtypeStruct((ACC_ROWS, FEAT), jnp.float32),
        ),
        scratch_types=[
            pltpu.VMEM((NCH, CHUNK), jnp.int32),
            pltpu.VMEM((CHUNK, FEAT), jnp.float32),
            pltpu.VMEM_SHARED((ACC_ROWS, FEAT), jnp.float32),
        ],
    )
    def k(dst_hbm, zeros_hbm, out_a, out_b, dst_v, ones_v, deg_sp):
        c = lax.axis_index("c")
        s = lax.axis_index("s")
        wid = c * SUBCORES + s
        pltpu.sync_copy(dst_hbm.at[wid], dst_v)

        def fill(i, carry):
            for kk in range(FEAT // 16):
                ones_v[i, pl.ds(kk * 16, 16)] = jnp.full((16,), 1.0,
                                                         jnp.float32)
            return carry

        lax.fori_loop(0, CHUNK, fill, 0)

        @pl.when(s == 0)
        def _():
            pltpu.sync_copy(zeros_hbm, deg_sp)

        plsc.subcore_barrier()

        def body(j, carry):
            pltpu.sync_copy(ones_v, deg_sp.at[dst_v.at[j]], add=True)
            return carry

        lax.fori_loop(0, NCH, body, 0)
        plsc.subcore_barrier()

        @pl.when((s == 0) & (c == 0))
        def _():
            pltpu.sync_copy(deg_sp, out_a)

        @pl.when((s == 0) & (c == 1))
        def _():
            pltpu.sync_copy(deg_sp, out_b)

    return k(dst_r, zeros_acc)


def _sc_scatter(y, src_r, dst_r, zeros_acc):
    @functools.partial(
        pl.kernel,
        mesh=_sc_mesh(),
        out_type=(
            jax.ShapeDtypeStruct((ACC_ROWS, FEAT), jnp.float32),
            jax.ShapeDtypeStruct((ACC_ROWS, FEAT), jnp.float32),
        ),
        scratch_types=[
            pltpu.VMEM((EPT,), jnp.int32),
            pltpu.VMEM((NCH, CHUNK), jnp.int32),
            pltpu.VMEM((NBUF, CHUNK, FEAT), jnp.float32),
            pltpu.VMEM_SHARED((ACC_ROWS, FEAT), jnp.float32),
        ] + [pltpu.SemaphoreType.DMA] * NBUF,
    )
    def k(y_hbm, src_hbm, dst_hbm, zeros_hbm, out_a, out_b,
          src_v, dst_v, rows_v, acc_sp, *sems):
        c = lax.axis_index("c")
        s = lax.axis_index("s")
        wid = c * SUBCORES + s
        pltpu.sync_copy(src_hbm.at[wid], src_v)
        pltpu.sync_copy(dst_hbm.at[wid], dst_v)

        @pl.when(s == 0)
        def _():
            pltpu.sync_copy(zeros_hbm, acc_sp)

        plsc.subcore_barrier()

        def _gather(j, b):
            pltpu.async_copy(y_hbm.at[src_v.at[pl.ds(j * CHUNK, CHUNK)]],
                             rows_v.at[b], sems[b])

        def _gwait(j, b):
            pltpu.make_async_copy(y_hbm.at[src_v.at[pl.ds(j * CHUNK, CHUNK)]],
                                  rows_v.at[b], sems[b]).wait()

        for b in range(NBUF):
            _gather(b, b)

        def body(g, carry):
            for b in range(NBUF):
                j = g * NBUF + b
                _gwait(j, b)
                pltpu.sync_copy(rows_v.at[b], acc_sp.at[dst_v.at[j]], add=True)

                @pl.when(g < NGRP - 1)
                def _():
                    _gather(j + NBUF, b)

            return carry

        lax.fori_loop(0, NGRP, body, 0)
        plsc.subcore_barrier()

        @pl.when((s == 0) & (c == 0))
        def _():
            pltpu.sync_copy(acc_sp, out_a)

        @pl.when((s == 0) & (c == 1))
        def _():
            pltpu.sync_copy(acc_sp, out_b)

    return k(y, src_r, dst_r, zeros_acc)


def _dinv_of(dega_ref, degb_ref):
    deg = dega_ref[:, 0] + degb_ref[:, 0] + 1.0
    return lax.rsqrt(deg)


def _tc_matmul(x, W0):
    def body(x_ref, w_ref, y_ref):
        y_ref[...] = jnp.dot(x_ref[...], w_ref[...],
                             preferred_element_type=jnp.float32)

    return pl.pallas_call(
        body,
        grid=(GRID,),
        in_specs=[
            pl.BlockSpec((RBLK, FEAT), lambda i: (i, 0)),
            pl.BlockSpec((FEAT, FEAT), lambda i: (0, 0)),
        ],
        out_specs=pl.BlockSpec((RBLK, FEAT), lambda i: (i, 0)),
        out_shape=jax.ShapeDtypeStruct((NODES, FEAT), jnp.float32),
    )(x, W0)


def _tc_scale(dega, degb, xw):
    def body(dega_ref, degb_ref, x_ref, y_ref):
        dinv = _dinv_of(dega_ref, degb_ref)
        y_ref[...] = dinv[:, None] * x_ref[...]

    return pl.pallas_call(
        body,
        grid=(GRID,),
        in_specs=[
            pl.BlockSpec((RBLK, FEAT), lambda i: (i, 0)),
            pl.BlockSpec((RBLK, FEAT), lambda i: (i, 0)),
            pl.BlockSpec((RBLK, FEAT), lambda i: (i, 0)),
        ],
        out_specs=pl.BlockSpec((RBLK, FEAT), lambda i: (i, 0)),
        out_shape=jax.ShapeDtypeStruct((NODES, FEAT), jnp.float32),
    )(dega, degb, xw)


def _tc_layer(dega, degb, acca, accb, yprev, brow, W):
    def body(dega_ref, degb_ref, aa_ref, ab_ref, y_ref, b_ref, w_ref, o_ref):
        dinv = _dinv_of(dega_ref, degb_ref)
        pre = dinv[:, None] * (aa_ref[...] + ab_ref[...] + y_ref[...]) + b_ref[...]
        h = jnp.maximum(pre, 0.0)
        o_ref[...] = dinv[:, None] * jnp.dot(
            h, w_ref[...], preferred_element_type=jnp.float32)

    return pl.pallas_call(
        body,
        grid=(GRID,),
        in_specs=[
            pl.BlockSpec((RBLK, FEAT), lambda i: (i, 0)),
            pl.BlockSpec((RBLK, FEAT), lambda i: (i, 0)),
            pl.BlockSpec((RBLK, FEAT), lambda i: (i, 0)),
            pl.BlockSpec((RBLK, FEAT), lambda i: (i, 0)),
            pl.BlockSpec((RBLK, FEAT), lambda i: (i, 0)),
            pl.BlockSpec((1, FEAT), lambda i: (0, 0)),
            pl.BlockSpec((FEAT, FEAT), lambda i: (0, 0)),
        ],
        out_specs=pl.BlockSpec((RBLK, FEAT), lambda i: (i, 0)),
        out_shape=jax.ShapeDtypeStruct((NODES, FEAT), jnp.float32),
    )(dega, degb, acca, accb, yprev, brow, W)


def _tc_final(dega, degb, acca, accb, yprev, brow, batch2d):
    def body(dega_ref, degb_ref, aa_ref, ab_ref, y_ref, b_ref, batch_ref, o_ref):
        dinv = _dinv_of(dega_ref, degb_ref)
        node = dinv[:, None] * (aa_ref[...] + ab_ref[...] + y_ref[...]) + b_ref[...]
        gids = lax.broadcasted_iota(jnp.int32, (1, GRAPHS), 1)
        onehot = (batch_ref[...] == gids).astype(jnp.float32)
        part = lax.dot_general(onehot, node, (((0,), (0,)), ((), ())),
                               preferred_element_type=jnp.float32)

        @pl.when(pl.program_id(0) == 0)
        def _():
            o_ref[...] = jnp.zeros_like(o_ref)

        o_ref[...] += part

    return pl.pallas_call(
        body,
        grid=(GRID,),
        in_specs=[
            pl.BlockSpec((RBLK, FEAT), lambda i: (i, 0)),
            pl.BlockSpec((RBLK, FEAT), lambda i: (i, 0)),
            pl.BlockSpec((RBLK, FEAT), lambda i: (i, 0)),
            pl.BlockSpec((RBLK, FEAT), lambda i: (i, 0)),
            pl.BlockSpec((RBLK, FEAT), lambda i: (i, 0)),
            pl.BlockSpec((1, FEAT), lambda i: (0, 0)),
            pl.BlockSpec((RBLK, 1), lambda i: (i, 0)),
        ],
        out_specs=pl.BlockSpec((GRAPHS, FEAT), lambda i: (0, 0)),
        out_shape=jax.ShapeDtypeStruct((GRAPHS, FEAT), jnp.float32),
    )(dega, degb, acca, accb, yprev, brow, batch2d)


def kernel(x, edge_index, batch, W0, b0, W1, b1, Wout, bout):
    npad = EDGES_PAD - EDGES
    pad_src = jnp.arange(npad, dtype=jnp.int32) * 7 % NODES
    src_r = jnp.concatenate([edge_index[1], pad_src]).reshape(TILES, EPT)
    junk = NODES + (jnp.arange(npad, dtype=jnp.int32) % JUNK_ROWS)
    dst_r = jnp.concatenate([edge_index[0], junk]).reshape(TILES, NCH, CHUNK)
    zeros_acc = jnp.zeros((ACC_ROWS, FEAT), jnp.float32)
    batch2d = batch.reshape(NODES, 1)
    b0r = b0.reshape(1, FEAT)
    b1r = b1.reshape(1, FEAT)
    boutr = bout.reshape(1, FEAT)

    xw0 = _tc_matmul(x, W0)
    dega, degb = _sc_degree(dst_r, zeros_acc)
    y0 = _tc_scale(dega, degb, xw0)
    a0, p0 = _sc_scatter(y0, src_r, dst_r, zeros_acc)
    y1 = _tc_layer(dega, degb, a0, p0, y0, b0r, W1)
    a1, p1 = _sc_scatter(y1, src_r, dst_r, zeros_acc)
    y2 = _tc_layer(dega, degb, a1, p1, y1, b1r, Wout)
    a2, p2 = _sc_scatter(y2, src_r, dst_r, zeros_acc)
    return _tc_final(dega, degb, a2, p2, y2, boutr, batch2d)


# speedup vs baseline: 1.0152x; 1.0152x over previous
import functools

import jax
import jax.numpy as jnp
from jax import lax
from jax.experimental import pallas as pl
from jax.experimental.pallas import tpu as pltpu
from jax.experimental.pallas import tpu_sc as plsc

NODES = 10000
JUNK_ROWS = 8
ACC_ROWS = NODES + JUNK_ROWS
FEAT = 128
EDGES = 320000
GRAPHS = 16

CORES = 2
SUBCORES = 16
TILES = CORES * SUBCORES

CHUNK = 104
NCH = 98
EPT = NCH * CHUNK
EDGES_PAD = EPT * TILES
NBUF = 2
NGRP = NCH // NBUF

RBLK = 2000
GRID = NODES // RBLK


def _sc_mesh():
    return plsc.VectorSubcoreMesh(core_axis_name="c", subcore_axis_name="s")


def _sc_degree(dst_r, zeros_acc):
    @functools.partial(
        pl.kernel,
        mesh=_sc_mesh(),
        out_type=(
            jax.ShapeDtypeStruct((ACC_ROWS, FEAT), jnp.float32),
            jax.ShapeDtypeStruct((ACC_ROWS, FEAT), jnp.float32),
        ),
        scratch_types=[
            pltpu.VMEM((NCH, CHUNK), jnp.int32),
            pltpu.VMEM((CHUNK, FEAT), jnp.float32),
            pltpu.VMEM_SHARED((ACC_ROWS, FEAT), jnp.float32),
        ],
    )
    def k(dst_hbm, zeros_hbm, out_a, out_b, dst_v, ones_v, deg_sp):
        c = lax.axis_index("c")
        s = lax.axis_index("s")
        wid = c * SUBCORES + s
        pltpu.sync_copy(dst_hbm.at[wid], dst_v)

        def fill(i, carry):
            for kk in range(FEAT // 16):
                ones_v[i, pl.ds(kk * 16, 16)] = jnp.full((16,), 1.0,
                                                         jnp.float32)
            return carry

        lax.fori_loop(0, CHUNK, fill, 0)

        @pl.when(s == 0)
        def _():
            pltpu.sync_copy(zeros_hbm, deg_sp)

        plsc.subcore_barrier()

        def body(j, carry):
            pltpu.sync_copy(ones_v, deg_sp.at[dst_v.at[j]], add=True)
            return carry

        lax.fori_loop(0, NCH, body, 0)
        plsc.subcore_barrier()

        @pl.when((s == 0) & (c == 0))
        def _():
            pltpu.sync_copy(deg_sp, out_a)

        @pl.when((s == 0) & (c == 1))
        def _():
            pltpu.sync_copy(deg_sp, out_b)

    return k(dst_r, zeros_acc)


def _sc_scatter(y, src_r, dst_r, zeros_acc):
    @functools.partial(
        pl.kernel,
        mesh=_sc_mesh(),
        out_type=(
            jax.ShapeDtypeStruct((ACC_ROWS, FEAT), jnp.float32),
            jax.ShapeDtypeStruct((ACC_ROWS, FEAT), jnp.float32),
        ),
        scratch_types=[
            pltpu.VMEM((EPT,), jnp.int32),
            pltpu.VMEM((NCH, CHUNK), jnp.int32),
            pltpu.VMEM((NBUF, CHUNK, FEAT), jnp.float32),
            pltpu.VMEM_SHARED((ACC_ROWS, FEAT), jnp.float32),
        ] + [pltpu.SemaphoreType.DMA] * NBUF,
    )
    def k(y_hbm, src_hbm, dst_hbm, zeros_hbm, out_a, out_b,
          src_v, dst_v, rows_v, acc_sp, *sems):
        c = lax.axis_index("c")
        s = lax.axis_index("s")
        wid = c * SUBCORES + s
        pltpu.sync_copy(src_hbm.at[wid], src_v)
        pltpu.sync_copy(dst_hbm.at[wid], dst_v)

        @pl.when(s == 0)
        def _():
            pltpu.sync_copy(zeros_hbm, acc_sp)

        plsc.subcore_barrier()

        def _gather(j, b):
            pltpu.async_copy(y_hbm.at[src_v.at[pl.ds(j * CHUNK, CHUNK)]],
                             rows_v.at[b], sems[b])

        def _gwait(j, b):
            pltpu.make_async_copy(y_hbm.at[src_v.at[pl.ds(j * CHUNK, CHUNK)]],
                                  rows_v.at[b], sems[b]).wait()

        for b in range(NBUF):
            _gather(b, b)

        def body(g, carry):
            for b in range(NBUF):
                j = g * NBUF + b
                _gwait(j, b)
                pltpu.sync_copy(rows_v.at[b], acc_sp.at[dst_v.at[j]], add=True)

                @pl.when(g < NGRP - 1)
                def _():
                    _gather(j + NBUF, b)

            return carry

        lax.fori_loop(0, NGRP, body, 0)
        plsc.subcore_barrier()

        @pl.when((s == 0) & (c == 0))
        def _():
            pltpu.sync_copy(acc_sp, out_a)

        @pl.when((s == 0) & (c == 1))
        def _():
            pltpu.sync_copy(acc_sp, out_b)

    return k(y, src_r, dst_r, zeros_acc)


def _dinv_of(dega_ref, degb_ref):
    deg = dega_ref[:, 0] + degb_ref[:, 0] + 1.0
    return lax.rsqrt(deg)


def _tc_first(dega, degb, x, W0):
    def body(dega_ref, degb_ref, x_ref, w_ref, y_ref):
        dinv = _dinv_of(dega_ref, degb_ref)
        xw = jnp.dot(x_ref[...], w_ref[...], preferred_element_type=jnp.float32)
        y_ref[...] = dinv[:, None] * xw

    return pl.pallas_call(
        body,
        grid=(GRID,),
        in_specs=[
            pl.BlockSpec((RBLK, FEAT), lambda i: (i, 0)),
            pl.BlockSpec((RBLK, FEAT), lambda i: (i, 0)),
            pl.BlockSpec((RBLK, FEAT), lambda i: (i, 0)),
            pl.BlockSpec((FEAT, FEAT), lambda i: (0, 0)),
        ],
        out_specs=pl.BlockSpec((RBLK, FEAT), lambda i: (i, 0)),
        out_shape=jax.ShapeDtypeStruct((NODES, FEAT), jnp.float32),
    )(dega, degb, x, W0)


def _tc_layer(dega, degb, acca, accb, yprev, brow, W):
    def body(dega_ref, degb_ref, aa_ref, ab_ref, y_ref, b_ref, w_ref, o_ref):
        dinv = _dinv_of(dega_ref, degb_ref)
        pre = dinv[:, None] * (aa_ref[...] + ab_ref[...] + y_ref[...]) + b_ref[...]
        h = jnp.maximum(pre, 0.0)
        o_ref[...] = dinv[:, None] * jnp.dot(
            h, w_ref[...], preferred_element_type=jnp.float32)

    return pl.pallas_call(
        body,
        grid=(GRID,),
        in_specs=[
            pl.BlockSpec((RBLK, FEAT), lambda i: (i, 0)),
            pl.BlockSpec((RBLK, FEAT), lambda i: (i, 0)),
            pl.BlockSpec((RBLK, FEAT), lambda i: (i, 0)),
            pl.BlockSpec((RBLK, FEAT), lambda i: (i, 0)),
            pl.BlockSpec((RBLK, FEAT), lambda i: (i, 0)),
            pl.BlockSpec((1, FEAT), lambda i: (0, 0)),
            pl.BlockSpec((FEAT, FEAT), lambda i: (0, 0)),
        ],
        out_specs=pl.BlockSpec((RBLK, FEAT), lambda i: (i, 0)),
        out_shape=jax.ShapeDtypeStruct((NODES, FEAT), jnp.float32),
    )(dega, degb, acca, accb, yprev, brow, W)


def _tc_final(dega, degb, acca, accb, yprev, brow, batch2d):
    def body(dega_ref, degb_ref, aa_ref, ab_ref, y_ref, b_ref, batch_ref, o_ref):
        dinv = _dinv_of(dega_ref, degb_ref)
        node = dinv[:, None] * (aa_ref[...] + ab_ref[...] + y_ref[...]) + b_ref[...]
        gids = lax.broadcasted_iota(jnp.int32, (1, GRAPHS), 1)
        onehot = (batch_ref[...] == gids).astype(jnp.float32)
        part = lax.dot_general(onehot, node, (((0,), (0,)), ((), ())),
                               preferred_element_type=jnp.float32)

        @pl.when(pl.program_id(0) == 0)
        def _():
            o_ref[...] = jnp.zeros_like(o_ref)

        o_ref[...] += part

    return pl.pallas_call(
        body,
        grid=(GRID,),
        in_specs=[
            pl.BlockSpec((RBLK, FEAT), lambda i: (i, 0)),
            pl.BlockSpec((RBLK, FEAT), lambda i: (i, 0)),
            pl.BlockSpec((RBLK, FEAT), lambda i: (i, 0)),
            pl.BlockSpec((RBLK, FEAT), lambda i: (i, 0)),
            pl.BlockSpec((RBLK, FEAT), lambda i: (i, 0)),
            pl.BlockSpec((1, FEAT), lambda i: (0, 0)),
            pl.BlockSpec((RBLK, 1), lambda i: (i, 0)),
        ],
        out_specs=pl.BlockSpec((GRAPHS, FEAT), lambda i: (0, 0)),
        out_shape=jax.ShapeDtypeStruct((GRAPHS, FEAT), jnp.float32),
    )(dega, degb, acca, accb, yprev, brow, batch2d)


def kernel(x, edge_index, batch, W0, b0, W1, b1, Wout, bout):
    npad = EDGES_PAD - EDGES
    pad_src = jnp.arange(npad, dtype=jnp.int32) * 7 % NODES
    src_r = jnp.concatenate([edge_index[1], pad_src]).reshape(TILES, EPT)
    junk = NODES + (jnp.arange(npad, dtype=jnp.int32) % JUNK_ROWS)
    dst_r = jnp.concatenate([edge_index[0], junk]).reshape(TILES, NCH, CHUNK)
    zeros_acc = jnp.zeros((ACC_ROWS, FEAT), jnp.float32)
    batch2d = batch.reshape(NODES, 1)
    b0r = b0.reshape(1, FEAT)
    b1r = b1.reshape(1, FEAT)
    boutr = bout.reshape(1, FEAT)

    dega, degb = _sc_degree(dst_r, zeros_acc)
    y0 = _tc_first(dega, degb, x, W0)
    a0, p0 = _sc_scatter(y0, src_r, dst_r, zeros_acc)
    y1 = _tc_layer(dega, degb, a0, p0, y0, b0r, W1)
    a1, p1 = _sc_scatter(y1, src_r, dst_r, zeros_acc)
    y2 = _tc_layer(dega, degb, a1, p1, y1, b1r, Wout)
    a2, p2 = _sc_scatter(y2, src_r, dst_r, zeros_acc)
    return _tc_final(dega, degb, a2, p2, y2, boutr, batch2d)


# speedup vs baseline: 1.0633x; 1.0473x over previous
import functools

import jax
import jax.numpy as jnp
from jax import lax
from jax.experimental import pallas as pl
from jax.experimental.pallas import tpu as pltpu
from jax.experimental.pallas import tpu_sc as plsc

NODES = 10000
JUNK_ROWS = 8
ACC_ROWS = NODES + JUNK_ROWS
FEAT = 128
EDGES = 320000
GRAPHS = 16

CORES = 2
SUBCORES = 16
TILES = CORES * SUBCORES

CHUNK = 80
NCH = 128
EPT = NCH * CHUNK
EDGES_PAD = EPT * TILES
NBUF = 4
NGRP = NCH // NBUF
NIGRP = NCH // 8

RBLK = 2000
GRID = NODES // RBLK


def _sc_mesh():
    return plsc.VectorSubcoreMesh(core_axis_name="c", subcore_axis_name="s")


def _sc_degree(dst_r, zeros_acc):
    @functools.partial(
        pl.kernel,
        mesh=_sc_mesh(),
        out_type=(
            jax.ShapeDtypeStruct((ACC_ROWS, FEAT), jnp.float32),
            jax.ShapeDtypeStruct((ACC_ROWS, FEAT), jnp.float32),
        ),
        scratch_types=[
            pltpu.VMEM((NCH, CHUNK), jnp.int32),
            pltpu.VMEM((CHUNK, FEAT), jnp.float32),
            pltpu.VMEM_SHARED((ACC_ROWS, FEAT), jnp.float32),
        ],
    )
    def k(dst_hbm, zeros_hbm, out_a, out_b, dst_v, ones_v, deg_sp):
        c = lax.axis_index("c")
        s = lax.axis_index("s")
        wid = c * SUBCORES + s
        pltpu.sync_copy(dst_hbm.at[pl.ds(wid * NCH, NCH)], dst_v)

        def fill(i, carry):
            for kk in range(FEAT // 16):
                ones_v[i, pl.ds(kk * 16, 16)] = jnp.full((16,), 1.0,
                                                         jnp.float32)
            return carry

        lax.fori_loop(0, CHUNK, fill, 0)

        @pl.when(s == 0)
        def _():
            pltpu.sync_copy(zeros_hbm, deg_sp)

        plsc.subcore_barrier()

        def body(j, carry):
            pltpu.sync_copy(ones_v, deg_sp.at[dst_v.at[j]], add=True)
            return carry

        lax.fori_loop(0, NCH, body, 0)
        plsc.subcore_barrier()

        @pl.when((s == 0) & (c == 0))
        def _():
            pltpu.sync_copy(deg_sp, out_a)

        @pl.when((s == 0) & (c == 1))
        def _():
            pltpu.sync_copy(deg_sp, out_b)

    return k(dst_r, zeros_acc)


def _sc_scatter(y, src_r, dst_r, zeros_acc):
    @functools.partial(
        pl.kernel,
        mesh=_sc_mesh(),
        out_type=(
            jax.ShapeDtypeStruct((ACC_ROWS, FEAT), jnp.float32),
            jax.ShapeDtypeStruct((ACC_ROWS, FEAT), jnp.float32),
        ),
        scratch_types=[
            pltpu.VMEM((2, 8, CHUNK), jnp.int32),
            pltpu.VMEM((2, 8, CHUNK), jnp.int32),
            pltpu.VMEM((NBUF, CHUNK, FEAT), jnp.float32),
            pltpu.VMEM_SHARED((ACC_ROWS, FEAT), jnp.float32),
        ] + [pltpu.SemaphoreType.DMA] * (2 * NBUF + 2),
    )
    def k(y_hbm, src_hbm, dst_hbm, zeros_hbm, out_a, out_b,
          sbuf, dbuf, rows_v, acc_sp, *sems):
        gsems = sems[:NBUF]
        ssems = sems[NBUF:2 * NBUF]
        isem_s = sems[2 * NBUF]
        isem_d = sems[2 * NBUF + 1]
        c = lax.axis_index("c")
        s = lax.axis_index("s")
        wid = c * SUBCORES + s

        tbase = wid * NCH
        pltpu.sync_copy(src_hbm.at[pl.ds(tbase, 8)], sbuf.at[0])
        pltpu.sync_copy(dst_hbm.at[pl.ds(tbase, 8)], dbuf.at[0])

        @pl.when(s == 0)
        def _():
            pltpu.sync_copy(zeros_hbm, acc_sp)

        plsc.subcore_barrier()

        for b in range(NBUF - 1):
            pltpu.async_copy(y_hbm.at[sbuf.at[0, b]], rows_v.at[b], gsems[b])

        def body(h, carry):
            base = h * 16
            for r in range(16):
                j = base + r
                b = r % 4

                if r == 0:
                    pltpu.async_copy(src_hbm.at[pl.ds(tbase + base + 8, 8)],
                                     sbuf.at[1], isem_s)
                    pltpu.async_copy(dst_hbm.at[pl.ds(tbase + base + 8, 8)],
                                     dbuf.at[1], isem_d)
                if r == 5:
                    pltpu.make_async_copy(src_hbm.at[pl.ds(tbase + base + 8, 8)],
                                          sbuf.at[1], isem_s).wait()
                    pltpu.make_async_copy(dst_hbm.at[pl.ds(tbase + base + 8, 8)],
                                          dbuf.at[1], isem_d).wait()
                if r == 8:
                    @pl.when(h < NCH // 16 - 1)
                    def _():
                        pltpu.async_copy(
                            src_hbm.at[pl.ds(tbase + base + 16, 8)],
                            sbuf.at[0], isem_s)
                        pltpu.async_copy(
                            dst_hbm.at[pl.ds(tbase + base + 16, 8)],
                            dbuf.at[0], isem_d)
                if r == 13:
                    @pl.when(h < NCH // 16 - 1)
                    def _():
                        pltpu.make_async_copy(
                            src_hbm.at[pl.ds(tbase + base + 16, 8)],
                            sbuf.at[0], isem_s).wait()
                        pltpu.make_async_copy(
                            dst_hbm.at[pl.ds(tbase + base + 16, 8)],
                            dbuf.at[0], isem_d).wait()

                pltpu.make_async_copy(y_hbm.at[sbuf.at[0, 0]], rows_v.at[b],
                                      gsems[b]).wait()
                pltpu.async_copy(rows_v.at[b],
                                 acc_sp.at[dbuf.at[r // 8, r % 8]],
                                 ssems[b], add=True)

                bn = (b + 3) % NBUF
                slot = ((r + 3) // 8) % 2
                row = (r + 3) % 8

                def _issue():
                    pltpu.async_copy(y_hbm.at[sbuf.at[slot, row]],
                                     rows_v.at[bn], gsems[bn])

                def _wait_prev():
                    pltpu.make_async_copy(rows_v.at[bn],
                                          acc_sp.at[dbuf.at[0, 0]],
                                          ssems[bn]).wait()

                if r == 0:
                    @pl.when(h >= 1)
                    def _():
                        _wait_prev()

                    _issue()
                elif r >= 13:
                    @pl.when(h < NCH // 16 - 1)
                    def _():
                        _wait_prev()
                        _issue()
                else:
                    _wait_prev()
                    _issue()

            return carry

        lax.fori_loop(0, NCH // 16, body, 0)

        for b in range(NBUF):
            pltpu.make_async_copy(rows_v.at[b], acc_sp.at[dbuf.at[0, 0]],
                                  ssems[b]).wait()

        plsc.subcore_barrier()

        @pl.when((s == 0) & (c == 0))
        def _():
            pltpu.sync_copy(acc_sp, out_a)

        @pl.when((s == 0) & (c == 1))
        def _():
            pltpu.sync_copy(acc_sp, out_b)

    return k(y, src_r, dst_r, zeros_acc)


def _dinv_of(dega_ref, degb_ref):
    deg = dega_ref[:, 0] + degb_ref[:, 0] + 1.0
    return lax.rsqrt(deg)


def _tc_first(dega, degb, x, W0):
    def body(dega_ref, degb_ref, x_ref, w_ref, y_ref):
        dinv = _dinv_of(dega_ref, degb_ref)
        xw = jnp.dot(x_ref[...], w_ref[...], preferred_element_type=jnp.float32)
        y_ref[...] = dinv[:, None] * xw

    return pl.pallas_call(
        body,
        grid=(GRID,),
        in_specs=[
            pl.BlockSpec((RBLK, FEAT), lambda i: (i, 0)),
            pl.BlockSpec((RBLK, FEAT), lambda i: (i, 0)),
            pl.BlockSpec((RBLK, FEAT), lambda i: (i, 0)),
            pl.BlockSpec((FEAT, FEAT), lambda i: (0, 0)),
        ],
        out_specs=pl.BlockSpec((RBLK, FEAT), lambda i: (i, 0)),
        out_shape=jax.ShapeDtypeStruct((NODES, FEAT), jnp.float32),
    )(dega, degb, x, W0)


def _tc_layer(dega, degb, acca, accb, yprev, brow, W):
    def body(dega_ref, degb_ref, aa_ref, ab_ref, y_ref, b_ref, w_ref, o_ref):
        dinv = _dinv_of(dega_ref, degb_ref)
        pre = dinv[:, None] * (aa_ref[...] + ab_ref[...] + y_ref[...]) + b_ref[...]
        h = jnp.maximum(pre, 0.0)
        o_ref[...] = dinv[:, None] * jnp.dot(
            h, w_ref[...], preferred_element_type=jnp.float32)

    return pl.pallas_call(
        body,
        grid=(GRID,),
        in_specs=[
            pl.BlockSpec((RBLK, FEAT), lambda i: (i, 0)),
            pl.BlockSpec((RBLK, FEAT), lambda i: (i, 0)),
            pl.BlockSpec((RBLK, FEAT), lambda i: (i, 0)),
            pl.BlockSpec((RBLK, FEAT), lambda i: (i, 0)),
            pl.BlockSpec((RBLK, FEAT), lambda i: (i, 0)),
            pl.BlockSpec((1, FEAT), lambda i: (0, 0)),
            pl.BlockSpec((FEAT, FEAT), lambda i: (0, 0)),
        ],
        out_specs=pl.BlockSpec((RBLK, FEAT), lambda i: (i, 0)),
        out_shape=jax.ShapeDtypeStruct((NODES, FEAT), jnp.float32),
    )(dega, degb, acca, accb, yprev, brow, W)


def _tc_final(dega, degb, acca, accb, yprev, brow, batch2d):
    def body(dega_ref, degb_ref, aa_ref, ab_ref, y_ref, b_ref, batch_ref, o_ref):
        dinv = _dinv_of(dega_ref, degb_ref)
        node = dinv[:, None] * (aa_ref[...] + ab_ref[...] + y_ref[...]) + b_ref[...]
        gids = lax.broadcasted_iota(jnp.int32, (1, GRAPHS), 1)
        onehot = (batch_ref[...] == gids).astype(jnp.float32)
        part = lax.dot_general(onehot, node, (((0,), (0,)), ((), ())),
                               preferred_element_type=jnp.float32)

        @pl.when(pl.program_id(0) == 0)
        def _():
            o_ref[...] = jnp.zeros_like(o_ref)

        o_ref[...] += part

    return pl.pallas_call(
        body,
        grid=(GRID,),
        in_specs=[
            pl.BlockSpec((RBLK, FEAT), lambda i: (i, 0)),
            pl.BlockSpec((RBLK, FEAT), lambda i: (i, 0)),
            pl.BlockSpec((RBLK, FEAT), lambda i: (i, 0)),
            pl.BlockSpec((RBLK, FEAT), lambda i: (i, 0)),
            pl.BlockSpec((RBLK, FEAT), lambda i: (i, 0)),
            pl.BlockSpec((1, FEAT), lambda i: (0, 0)),
            pl.BlockSpec((RBLK, 1), lambda i: (i, 0)),
        ],
        out_specs=pl.BlockSpec((GRAPHS, FEAT), lambda i: (0, 0)),
        out_shape=jax.ShapeDtypeStruct((GRAPHS, FEAT), jnp.float32),
    )(dega, degb, acca, accb, yprev, brow, batch2d)


def kernel(x, edge_index, batch, W0, b0, W1, b1, Wout, bout):
    npad = EDGES_PAD - EDGES
    pad_src = jnp.arange(npad, dtype=jnp.int32) * 7 % NODES
    src_r = jnp.concatenate([edge_index[1], pad_src]).reshape(TILES * NCH, CHUNK)
    junk = NODES + (jnp.arange(npad, dtype=jnp.int32) % JUNK_ROWS)
    dst_r = jnp.concatenate([edge_index[0], junk]).reshape(TILES * NCH, CHUNK)
    zeros_acc = jnp.zeros((ACC_ROWS, FEAT), jnp.float32)
    batch2d = batch.reshape(NODES, 1)
    b0r = b0.reshape(1, FEAT)
    b1r = b1.reshape(1, FEAT)
    boutr = bout.reshape(1, FEAT)

    dega, degb = _sc_degree(dst_r, zeros_acc)
    y0 = _tc_first(dega, degb, x, W0)
    a0, p0 = _sc_scatter(y0, src_r, dst_r, zeros_acc)
    y1 = _tc_layer(dega, degb, a0, p0, y0, b0r, W1)
    a1, p1 = _sc_scatter(y1, src_r, dst_r, zeros_acc)
    y2 = _tc_layer(dega, degb, a1, p1, y1, b1r, Wout)
    a2, p2 = _sc_scatter(y2, src_r, dst_r, zeros_acc)
    return _tc_final(dega, degb, a2, p2, y2, boutr, batch2d)


# speedup vs baseline: 1.0663x; 1.0029x over previous
import functools

import jax
import jax.numpy as jnp
from jax import lax
from jax.experimental import pallas as pl
from jax.experimental.pallas import tpu as pltpu
from jax.experimental.pallas import tpu_sc as plsc

NODES = 10000
JUNK_ROWS = 8
ACC_ROWS = NODES + JUNK_ROWS
FEAT = 128
EDGES = 320000
GRAPHS = 16

CORES = 2
SUBCORES = 16
TILES = CORES * SUBCORES

CHUNK = 80
NCH = 128
EPT = NCH * CHUNK
EDGES_PAD = EPT * TILES
NBUF = 4
NGRP = NCH // NBUF
NIGRP = NCH // 8

RBLK = 2000
GRID = NODES // RBLK


def _sc_mesh():
    return plsc.VectorSubcoreMesh(core_axis_name="c", subcore_axis_name="s")


def _sc_degree(dst_r, zeros_acc):
    @functools.partial(
        pl.kernel,
        mesh=_sc_mesh(),
        out_type=(
            jax.ShapeDtypeStruct((ACC_ROWS, FEAT), jnp.float32),
            jax.ShapeDtypeStruct((ACC_ROWS, FEAT), jnp.float32),
        ),
        scratch_types=[
            pltpu.VMEM((NCH, CHUNK), jnp.int32),
            pltpu.VMEM((CHUNK, FEAT), jnp.float32),
            pltpu.VMEM_SHARED((ACC_ROWS, FEAT), jnp.float32),
            pltpu.SemaphoreType.DMA,
        ],
    )
    def k(dst_hbm, zeros_hbm, out_a, out_b, dst_v, ones_v, deg_sp, dsem):
        c = lax.axis_index("c")
        s = lax.axis_index("s")
        wid = c * SUBCORES + s
        pltpu.sync_copy(dst_hbm.at[pl.ds(wid * NCH, NCH)], dst_v)

        def fill(i, carry):
            for kk in range(FEAT // 16):
                ones_v[i, pl.ds(kk * 16, 16)] = jnp.full((16,), 1.0,
                                                         jnp.float32)
            return carry

        lax.fori_loop(0, CHUNK, fill, 0)

        @pl.when(s == 0)
        def _():
            pltpu.sync_copy(zeros_hbm, deg_sp)

        plsc.subcore_barrier()

        def group(gi, carry):
            def fire(r, c2):
                pltpu.async_copy(ones_v, deg_sp.at[dst_v.at[gi * 16 + r]],
                                 dsem, add=True)
                return c2

            lax.fori_loop(0, 16, fire, 0)

            def drain(r, c2):
                pltpu.make_async_copy(ones_v, deg_sp.at[dst_v.at[0]],
                                      dsem).wait()
                return c2

            lax.fori_loop(0, 16, drain, 0)
            return carry

        lax.fori_loop(0, NCH // 16, group, 0)
        plsc.subcore_barrier()

        @pl.when((s == 0) & (c == 0))
        def _():
            pltpu.sync_copy(deg_sp, out_a)

        @pl.when((s == 0) & (c == 1))
        def _():
            pltpu.sync_copy(deg_sp, out_b)

    return k(dst_r, zeros_acc)


def _sc_scatter(y, src_r, dst_r, zeros_acc):
    @functools.partial(
        pl.kernel,
        mesh=_sc_mesh(),
        out_type=(
            jax.ShapeDtypeStruct((ACC_ROWS, FEAT), jnp.float32),
            jax.ShapeDtypeStruct((ACC_ROWS, FEAT), jnp.float32),
        ),
        scratch_types=[
            pltpu.VMEM((2, 8, CHUNK), jnp.int32),
            pltpu.VMEM((2, 8, CHUNK), jnp.int32),
            pltpu.VMEM((NBUF, CHUNK, FEAT), jnp.float32),
            pltpu.VMEM_SHARED((ACC_ROWS, FEAT), jnp.float32),
        ] + [pltpu.SemaphoreType.DMA] * (2 * NBUF + 2),
    )
    def k(y_hbm, src_hbm, dst_hbm, zeros_hbm, out_a, out_b,
          sbuf, dbuf, rows_v, acc_sp, *sems):
        gsems = sems[:NBUF]
        ssems = sems[NBUF:2 * NBUF]
        isem_s = sems[2 * NBUF]
        isem_d = sems[2 * NBUF + 1]
        c = lax.axis_index("c")
        s = lax.axis_index("s")
        wid = c * SUBCORES + s

        tbase = wid * NCH
        pltpu.sync_copy(src_hbm.at[pl.ds(tbase, 8)], sbuf.at[0])
        pltpu.sync_copy(dst_hbm.at[pl.ds(tbase, 8)], dbuf.at[0])

        @pl.when(s == 0)
        def _():
            pltpu.sync_copy(zeros_hbm, acc_sp)

        plsc.subcore_barrier()

        for b in range(NBUF - 1):
            pltpu.async_copy(y_hbm.at[sbuf.at[0, b]], rows_v.at[b], gsems[b])

        def body(h, carry):
            base = h * 16
            for r in range(16):
                j = base + r
                b = r % 4

                if r == 0:
                    pltpu.async_copy(src_hbm.at[pl.ds(tbase + base + 8, 8)],
                                     sbuf.at[1], isem_s)
                    pltpu.async_copy(dst_hbm.at[pl.ds(tbase + base + 8, 8)],
                                     dbuf.at[1], isem_d)
                if r == 5:
                    pltpu.make_async_copy(src_hbm.at[pl.ds(tbase + base + 8, 8)],
                                          sbuf.at[1], isem_s).wait()
                    pltpu.make_async_copy(dst_hbm.at[pl.ds(tbase + base + 8, 8)],
                                          dbuf.at[1], isem_d).wait()
                if r == 8:
                    @pl.when(h < NCH // 16 - 1)
                    def _():
                        pltpu.async_copy(
                            src_hbm.at[pl.ds(tbase + base + 16, 8)],
                            sbuf.at[0], isem_s)
                        pltpu.async_copy(
                            dst_hbm.at[pl.ds(tbase + base + 16, 8)],
                            dbuf.at[0], isem_d)
                if r == 13:
                    @pl.when(h < NCH // 16 - 1)
                    def _():
                        pltpu.make_async_copy(
                            src_hbm.at[pl.ds(tbase + base + 16, 8)],
                            sbuf.at[0], isem_s).wait()
                        pltpu.make_async_copy(
                            dst_hbm.at[pl.ds(tbase + base + 16, 8)],
                            dbuf.at[0], isem_d).wait()

                pltpu.make_async_copy(y_hbm.at[sbuf.at[0, 0]], rows_v.at[b],
                                      gsems[b]).wait()
                pltpu.async_copy(rows_v.at[b],
                                 acc_sp.at[dbuf.at[r // 8, r % 8]],
                                 ssems[b], add=True)

                bn = (b + 3) % NBUF
                slot = ((r + 3) // 8) % 2
                row = (r + 3) % 8

                def _issue():
                    pltpu.async_copy(y_hbm.at[sbuf.at[slot, row]],
                                     rows_v.at[bn], gsems[bn])

                def _wait_prev():
                    pltpu.make_async_copy(rows_v.at[bn],
                                          acc_sp.at[dbuf.at[0, 0]],
                                          ssems[bn]).wait()

                if r == 0:
                    @pl.when(h >= 1)
                    def _():
                        _wait_prev()

                    _issue()
                elif r >= 13:
                    @pl.when(h < NCH // 16 - 1)
                    def _():
                        _wait_prev()
                        _issue()
                else:
                    _wait_prev()
                    _issue()

            return carry

        lax.fori_loop(0, NCH // 16, body, 0)

        for b in range(NBUF):
            pltpu.make_async_copy(rows_v.at[b], acc_sp.at[dbuf.at[0, 0]],
                                  ssems[b]).wait()

        plsc.subcore_barrier()

        @pl.when((s == 0) & (c == 0))
        def _():
            pltpu.sync_copy(acc_sp, out_a)

        @pl.when((s == 0) & (c == 1))
        def _():
            pltpu.sync_copy(acc_sp, out_b)

    return k(y, src_r, dst_r, zeros_acc)


def _dinv_of(dega_ref, degb_ref):
    deg = dega_ref[:, 0] + degb_ref[:, 0] + 1.0
    return lax.rsqrt(deg)


def _tc_first(dega, degb, x, W0):
    def body(dega_ref, degb_ref, x_ref, w_ref, y_ref):
        dinv = _dinv_of(dega_ref, degb_ref)
        xw = jnp.dot(x_ref[...], w_ref[...], preferred_element_type=jnp.float32)
        y_ref[...] = dinv[:, None] * xw

    return pl.pallas_call(
        body,
        grid=(GRID,),
        in_specs=[
            pl.BlockSpec((RBLK, FEAT), lambda i: (i, 0)),
            pl.BlockSpec((RBLK, FEAT), lambda i: (i, 0)),
            pl.BlockSpec((RBLK, FEAT), lambda i: (i, 0)),
            pl.BlockSpec((FEAT, FEAT), lambda i: (0, 0)),
        ],
        out_specs=pl.BlockSpec((RBLK, FEAT), lambda i: (i, 0)),
        out_shape=jax.ShapeDtypeStruct((NODES, FEAT), jnp.float32),
    )(dega, degb, x, W0)


def _tc_layer(dega, degb, acca, accb, yprev, brow, W):
    def body(dega_ref, degb_ref, aa_ref, ab_ref, y_ref, b_ref, w_ref, o_ref):
        dinv = _dinv_of(dega_ref, degb_ref)
        pre = dinv[:, None] * (aa_ref[...] + ab_ref[...] + y_ref[...]) + b_ref[...]
        h = jnp.maximum(pre, 0.0)
        o_ref[...] = dinv[:, None] * jnp.dot(
            h, w_ref[...], preferred_element_type=jnp.float32)

    return pl.pallas_call(
        body,
        grid=(GRID,),
        in_specs=[
            pl.BlockSpec((RBLK, FEAT), lambda i: (i, 0)),
            pl.BlockSpec((RBLK, FEAT), lambda i: (i, 0)),
            pl.BlockSpec((RBLK, FEAT), lambda i: (i, 0)),
            pl.BlockSpec((RBLK, FEAT), lambda i: (i, 0)),
            pl.BlockSpec((RBLK, FEAT), lambda i: (i, 0)),
            pl.BlockSpec((1, FEAT), lambda i: (0, 0)),
            pl.BlockSpec((FEAT, FEAT), lambda i: (0, 0)),
        ],
        out_specs=pl.BlockSpec((RBLK, FEAT), lambda i: (i, 0)),
        out_shape=jax.ShapeDtypeStruct((NODES, FEAT), jnp.float32),
    )(dega, degb, acca, accb, yprev, brow, W)


def _tc_final(dega, degb, acca, accb, yprev, brow, batch2d):
    def body(dega_ref, degb_ref, aa_ref, ab_ref, y_ref, b_ref, batch_ref, o_ref):
        dinv = _dinv_of(dega_ref, degb_ref)
        node = dinv[:, None] * (aa_ref[...] + ab_ref[...] + y_ref[...]) + b_ref[...]
        gids = lax.broadcasted_iota(jnp.int32, (1, GRAPHS), 1)
        onehot = (batch_ref[...] == gids).astype(jnp.float32)
        part = lax.dot_general(onehot, node, (((0,), (0,)), ((), ())),
                               preferred_element_type=jnp.float32)

        @pl.when(pl.program_id(0) == 0)
        def _():
            o_ref[...] = jnp.zeros_like(o_ref)

        o_ref[...] += part

    return pl.pallas_call(
        body,
        grid=(GRID,),
        in_specs=[
            pl.BlockSpec((RBLK, FEAT), lambda i: (i, 0)),
            pl.BlockSpec((RBLK, FEAT), lambda i: (i, 0)),
            pl.BlockSpec((RBLK, FEAT), lambda i: (i, 0)),
            pl.BlockSpec((RBLK, FEAT), lambda i: (i, 0)),
            pl.BlockSpec((RBLK, FEAT), lambda i: (i, 0)),
            pl.BlockSpec((1, FEAT), lambda i: (0, 0)),
            pl.BlockSpec((RBLK, 1), lambda i: (i, 0)),
        ],
        out_specs=pl.BlockSpec((GRAPHS, FEAT), lambda i: (0, 0)),
        out_shape=jax.ShapeDtypeStruct((GRAPHS, FEAT), jnp.float32),
    )(dega, degb, acca, accb, yprev, brow, batch2d)


def kernel(x, edge_index, batch, W0, b0, W1, b1, Wout, bout):
    npad = EDGES_PAD - EDGES
    pad_src = jnp.arange(npad, dtype=jnp.int32) * 7 % NODES
    src_r = jnp.concatenate([edge_index[1], pad_src]).reshape(TILES * NCH, CHUNK)
    junk = NODES + (jnp.arange(npad, dtype=jnp.int32) % JUNK_ROWS)
    dst_r = jnp.concatenate([edge_index[0], junk]).reshape(TILES * NCH, CHUNK)
    zeros_acc = jnp.zeros((ACC_ROWS, FEAT), jnp.float32)
    batch2d = batch.reshape(NODES, 1)
    b0r = b0.reshape(1, FEAT)
    b1r = b1.reshape(1, FEAT)
    boutr = bout.reshape(1, FEAT)

    dega, degb = _sc_degree(dst_r, zeros_acc)
    y0 = _tc_first(dega, degb, x, W0)
    a0, p0 = _sc_scatter(y0, src_r, dst_r, zeros_acc)
    y1 = _tc_layer(dega, degb, a0, p0, y0, b0r, W1)
    a1, p1 = _sc_scatter(y1, src_r, dst_r, zeros_acc)
    y2 = _tc_layer(dega, degb, a1, p1, y1, b1r, Wout)
    a2, p2 = _sc_scatter(y2, src_r, dst_r, zeros_acc)
    return _tc_final(dega, degb, a2, p2, y2, boutr, batch2d)
